# baseline (speedup 1.0000x reference)
import functools

import jax
import jax.numpy as jnp
from jax import lax
from jax.experimental import pallas as pl
from jax.experimental.pallas import tpu as pltpu
from jax.experimental.pallas import tpu_sc as plsc

B, L, V, H = 4, 2048, 30522, 768
EPS = 1e-12

NC, NS = 2, 16
NW = NC * NS
L_PER_W = L // NW
CHUNK = 32
N_CHUNK = B * L_PER_W // CHUNK
NBUF = 3
LANES = 16
NV = H // LANES


def _rsqrt(x):
    i = lax.bitcast_convert_type(x, jnp.int32)
    y = lax.bitcast_convert_type(jnp.int32(0x5F3759DF) - (i >> 1), jnp.float32)
    for _ in range(4):
        y = y * (1.5 - 0.5 * x * y * y)
    return y


def _body(ids_hbm, pos_hbm, tab_hbm, w_hbm, bias_hbm, out_hbm,
          idx_v, pos_v, g0, g1, g2, w_v, bias_v, m_s, i_s,
          gs0, gs1, gs2, ss0, ss1, ss2):
    cid = lax.axis_index("c")
    sid = lax.axis_index("s")
    wid = sid * NC + cid
    l0 = wid * L_PER_W

    bufs = (g0, g1, g2)
    gsems = (gs0, gs1, gs2)
    ssems = (ss0, ss1, ss2)

    for b in range(B):
        pltpu.sync_copy(ids_hbm.at[pl.ds(b * L + l0, L_PER_W)], idx_v.at[b])

    def idx_slice(j):
        b, h = j // 2, j % 2
        return idx_v.at[b, pl.ds(h * CHUNK, CHUNK)]

    gathers = [None] * N_CHUNK
    stores = [None] * N_CHUNK
    gathers[0] = pltpu.async_copy(tab_hbm.at[idx_slice(0)], bufs[0], gsems[0])
    gathers[1] = pltpu.async_copy(tab_hbm.at[idx_slice(1)], bufs[1], gsems[1])

    pltpu.sync_copy(pos_hbm.at[pl.ds(l0, L_PER_W)], pos_v)
    pltpu.sync_copy(w_hbm, w_v)
    pltpu.sync_copy(bias_hbm, bias_v)

    def compute(gbuf, j):
        h = j % 2

        def sum_body(rr, carry):
            r0 = rr * 2
            r1 = r0 + 1
            p0 = h * CHUNK + r0
            p1 = p0 + 1

            z = jnp.zeros((LANES,), jnp.float32)

            @plsc.parallel_loop(0, NV, 2, unroll=4,
                                carry=(z, z, z, z, z, z, z, z))
            def slice_body(k, accs):
                a0, a1, a2, a3, q0, q1, q2, q3 = accs
                sl_e = pl.ds(k * LANES, LANES)
                sl_o = pl.ds((k + 1) * LANES, LANES)
                va = gbuf[r0, sl_e] + pos_v[p0, sl_e]
                vb = gbuf[r1, sl_e] + pos_v[p1, sl_e]
                vc = gbuf[r0, sl_o] + pos_v[p0, sl_o]
                vd = gbuf[r1, sl_o] + pos_v[p1, sl_o]
                gbuf[r0, sl_e] = va
                gbuf[r1, sl_e] = vb
                gbuf[r0, sl_o] = vc
                gbuf[r1, sl_o] = vd
                return (a0 + va, a1 + vc, a2 + vb, a3 + vd,
                        q0 + va * va, q1 + vc * vc,
                        q2 + vb * vb, q3 + vd * vd)

            a0, a1, a2, a3, q0, q1, q2, q3 = slice_body
            s0 = jnp.sum(a0 + a1)
            s1 = jnp.sum(a2 + a3)
            t0 = jnp.sum(q0 + q1)
            t1 = jnp.sum(q2 + q3)
            mean0 = s0 * (1.0 / H)
            mean1 = s1 * (1.0 / H)
            var0 = jnp.maximum(t0 * (1.0 / H) - mean0 * mean0, 0.0)
            var1 = jnp.maximum(t1 * (1.0 / H) - mean1 * mean1, 0.0)
            m_s[r0] = mean0
            m_s[r1] = mean1
            i_s[r0] = _rsqrt(var0 + EPS)
            i_s[r1] = _rsqrt(var1 + EPS)
            return carry

        lax.fori_loop(0, CHUNK // 2, sum_body, 0)

        @plsc.parallel_loop(0, NV * (CHUNK // 8), 1, unroll=2)
        def norm_tile(i):
            k = i >> 2
            rg = i & 3
            sl = pl.ds(k * LANES, LANES)
            wv = w_v[sl]
            bv = bias_v[sl]
            r0 = rg * 8
            for dr in range(8):
                r = r0 + dr
                gbuf[r, sl] = (gbuf[r, sl] - m_s[r]) * i_s[r] * wv + bv

    for j in range(N_CHUNK):
        bi = j % NBUF
        gathers[j].wait()
        compute(bufs[bi], j)
        b, h = j // 2, j % 2
        row0 = b * L + l0 + h * CHUNK
        stores[j] = pltpu.async_copy(bufs[bi], out_hbm.at[pl.ds(row0, CHUNK)],
                                     ssems[bi])
        nj = j + 2
        if nj < N_CHUNK:
            nbi = nj % NBUF
            if nj - NBUF >= 0:
                stores[nj - NBUF].wait()
            gathers[nj] = pltpu.async_copy(
                tab_hbm.at[idx_slice(nj)], bufs[nbi], gsems[nbi])
    for j in range(N_CHUNK - NBUF, N_CHUNK):
        stores[j].wait()


def kernel(input_ids, attention_mask, word_embeddings, position_embeddings,
           ln_weight, ln_bias):
    del attention_mask
    ids_flat = input_ids.reshape(-1).astype(jnp.int32)
    mesh = plsc.VectorSubcoreMesh(
        core_axis_name="c", subcore_axis_name="s",
        num_cores=NC, num_subcores=NS)
    fn = functools.partial(
        pl.kernel,
        out_type=jax.ShapeDtypeStruct((B * L, H), jnp.float32),
        mesh=mesh,
        scratch_types=[
            pltpu.VMEM((B, L_PER_W), jnp.int32),
            pltpu.VMEM((L_PER_W, H), jnp.float32),
            pltpu.VMEM((CHUNK, H), jnp.float32),
            pltpu.VMEM((CHUNK, H), jnp.float32),
            pltpu.VMEM((CHUNK, H), jnp.float32),
            pltpu.VMEM((H,), jnp.float32),
            pltpu.VMEM((H,), jnp.float32),
            pltpu.SMEM((CHUNK,), jnp.float32),
            pltpu.SMEM((CHUNK,), jnp.float32),
            pltpu.SemaphoreType.DMA,
            pltpu.SemaphoreType.DMA,
            pltpu.SemaphoreType.DMA,
            pltpu.SemaphoreType.DMA,
            pltpu.SemaphoreType.DMA,
            pltpu.SemaphoreType.DMA,
        ],
        compiler_params=pltpu.CompilerParams(needs_layout_passes=False),
    )(_body)
    out = fn(ids_flat, position_embeddings, word_embeddings, ln_weight, ln_bias)
    return out.reshape(B, L, H)


# speedup vs baseline: 1.2132x; 1.2132x over previous
import functools

import jax
import jax.numpy as jnp
from jax import lax
from jax.experimental import pallas as pl
from jax.experimental.pallas import tpu as pltpu
from jax.experimental.pallas import tpu_sc as plsc

B, L, V, H = 4, 2048, 30522, 768
EPS = 1e-12

NC, NS = 2, 16
NW = NC * NS
L_PER_W = L // NW
CHUNK = 32
N_CHUNK = B * L_PER_W // CHUNK
NBUF = 3
LANES = 16
NV = H // LANES


def _rsqrt(x):
    i = lax.bitcast_convert_type(x, jnp.int32)
    y = lax.bitcast_convert_type(jnp.int32(0x5F3759DF) - (i >> 1), jnp.float32)
    for _ in range(4):
        y = y * (1.5 - 0.5 * x * y * y)
    return y


def _body(ids_hbm, pos_hbm, tab_hbm, w_hbm, bias_hbm, out_hbm,
          idx_v, pos_v, g0, g1, g2, w_v, bias_v, m_s, i_s,
          gs0, gs1, gs2, ss0, ss1, ss2):
    cid = lax.axis_index("c")
    sid = lax.axis_index("s")
    wid = sid * NC + cid
    l0 = wid * L_PER_W

    bufs = (g0, g1, g2)
    gsems = (gs0, gs1, gs2)
    ssems = (ss0, ss1, ss2)

    for b in range(B):
        pltpu.sync_copy(ids_hbm.at[pl.ds(b * L + l0, L_PER_W)], idx_v.at[b])

    def idx_slice(j):
        b, h = j // 2, j % 2
        return idx_v.at[b, pl.ds(h * CHUNK, CHUNK)]

    gathers = [None] * N_CHUNK
    stores = [None] * N_CHUNK
    gathers[0] = pltpu.async_copy(tab_hbm.at[idx_slice(0)], bufs[0], gsems[0])
    gathers[1] = pltpu.async_copy(tab_hbm.at[idx_slice(1)], bufs[1], gsems[1])

    pltpu.sync_copy(pos_hbm.at[pl.ds(l0, L_PER_W)], pos_v)
    pltpu.sync_copy(w_hbm, w_v)
    pltpu.sync_copy(bias_hbm, bias_v)

    def compute(gbuf, j):
        h = j % 2

        def sum_body(rr, carry):
            r0 = rr * 2
            r1 = r0 + 1
            p0 = h * CHUNK + r0
            p1 = p0 + 1

            z = jnp.zeros((LANES,), jnp.float32)

            @plsc.parallel_loop(0, NV, 2, unroll=4,
                                carry=(z, z, z, z, z, z, z, z))
            def slice_body(k, accs):
                a0, a1, a2, a3, q0, q1, q2, q3 = accs
                sl_e = pl.ds(k * LANES, LANES)
                sl_o = pl.ds((k + 1) * LANES, LANES)
                va = gbuf[r0, sl_e] + pos_v[p0, sl_e]
                vb = gbuf[r1, sl_e] + pos_v[p1, sl_e]
                vc = gbuf[r0, sl_o] + pos_v[p0, sl_o]
                vd = gbuf[r1, sl_o] + pos_v[p1, sl_o]
                gbuf[r0, sl_e] = va
                gbuf[r1, sl_e] = vb
                gbuf[r0, sl_o] = vc
                gbuf[r1, sl_o] = vd
                return (a0 + va, a1 + vc, a2 + vb, a3 + vd,
                        q0 + va * va, q1 + vc * vc,
                        q2 + vb * vb, q3 + vd * vd)

            a0, a1, a2, a3, q0, q1, q2, q3 = slice_body
            s0 = jnp.sum(a0 + a1)
            s1 = jnp.sum(a2 + a3)
            t0 = jnp.sum(q0 + q1)
            t1 = jnp.sum(q2 + q3)
            mean0 = s0 * (1.0 / H)
            mean1 = s1 * (1.0 / H)
            var0 = jnp.maximum(t0 * (1.0 / H) - mean0 * mean0, 0.0)
            var1 = jnp.maximum(t1 * (1.0 / H) - mean1 * mean1, 0.0)
            m_s[r0] = mean0
            m_s[r1] = mean1
            i_s[r0] = _rsqrt(var0 + EPS)
            i_s[r1] = _rsqrt(var1 + EPS)
            return carry

        lax.fori_loop(0, CHUNK // 2, sum_body, 0)

        def norm_slice(k, carry):
            sl = pl.ds(k * LANES, LANES)
            wv = w_v[sl]
            bv = bias_v[sl]

            def norm_rows(rg, c2):
                r0 = rg * 8
                for dr in range(8):
                    r = r0 + dr
                    gbuf[r, sl] = (gbuf[r, sl] - m_s[r]) * i_s[r] * wv + bv
                return c2

            lax.fori_loop(0, CHUNK // 8, norm_rows, 0)
            return carry

        lax.fori_loop(0, NV, norm_slice, 0)

    for j in range(N_CHUNK):
        bi = j % NBUF
        gathers[j].wait()
        compute(bufs[bi], j)
        b, h = j // 2, j % 2
        row0 = b * L + l0 + h * CHUNK
        stores[j] = pltpu.async_copy(bufs[bi], out_hbm.at[pl.ds(row0, CHUNK)],
                                     ssems[bi])
        nj = j + 2
        if nj < N_CHUNK:
            nbi = nj % NBUF
            if nj - NBUF >= 0:
                stores[nj - NBUF].wait()
            gathers[nj] = pltpu.async_copy(
                tab_hbm.at[idx_slice(nj)], bufs[nbi], gsems[nbi])
    for j in range(N_CHUNK - NBUF, N_CHUNK):
        stores[j].wait()


def kernel(input_ids, attention_mask, word_embeddings, position_embeddings,
           ln_weight, ln_bias):
    del attention_mask
    ids_flat = input_ids.reshape(-1).astype(jnp.int32)
    mesh = plsc.VectorSubcoreMesh(
        core_axis_name="c", subcore_axis_name="s",
        num_cores=NC, num_subcores=NS)
    fn = functools.partial(
        pl.kernel,
        out_type=jax.ShapeDtypeStruct((B * L, H), jnp.float32),
        mesh=mesh,
        scratch_types=[
            pltpu.VMEM((B, L_PER_W), jnp.int32),
            pltpu.VMEM((L_PER_W, H), jnp.float32),
            pltpu.VMEM((CHUNK, H), jnp.float32),
            pltpu.VMEM((CHUNK, H), jnp.float32),
            pltpu.VMEM((CHUNK, H), jnp.float32),
            pltpu.VMEM((H,), jnp.float32),
            pltpu.VMEM((H,), jnp.float32),
            pltpu.SMEM((CHUNK,), jnp.float32),
            pltpu.SMEM((CHUNK,), jnp.float32),
            pltpu.SemaphoreType.DMA,
            pltpu.SemaphoreType.DMA,
            pltpu.SemaphoreType.DMA,
            pltpu.SemaphoreType.DMA,
            pltpu.SemaphoreType.DMA,
            pltpu.SemaphoreType.DMA,
        ],
        compiler_params=pltpu.CompilerParams(needs_layout_passes=False),
    )(_body)
    out = fn(ids_flat, position_embeddings, word_embeddings, ln_weight, ln_bias)
    return out.reshape(B, L, H)


# speedup vs baseline: 1.2421x; 1.0238x over previous
import functools

import jax
import jax.numpy as jnp
from jax import lax
from jax.experimental import pallas as pl
from jax.experimental.pallas import tpu as pltpu
from jax.experimental.pallas import tpu_sc as plsc

B, L, V, H = 4, 2048, 30522, 768
EPS = 1e-12

NC, NS = 2, 16
NW = NC * NS
L_PER_W = L // NW
CHUNK = 32
N_CHUNK = B * L_PER_W // CHUNK
NBUF = 3
LANES = 16
NV = H // LANES


def _rsqrt(x):
    i = lax.bitcast_convert_type(x, jnp.int32)
    y = lax.bitcast_convert_type(jnp.int32(0x5F3759DF) - (i >> 1), jnp.float32)
    for _ in range(3):
        y = y * (1.5 - 0.5 * x * y * y)
    return y


def _body(ids_hbm, pos_hbm, tab_hbm, w_hbm, bias_hbm, out_hbm,
          idx_v, pos_v, g0, g1, g2, w_v, bias_v, m_s, i_s,
          gs0, gs1, gs2, ss0, ss1, ss2):
    cid = lax.axis_index("c")
    sid = lax.axis_index("s")
    wid = sid * NC + cid
    l0 = wid * L_PER_W

    bufs = (g0, g1, g2)
    gsems = (gs0, gs1, gs2)
    ssems = (ss0, ss1, ss2)

    def idx_slice(j):
        b, h = j // 2, j % 2
        return idx_v.at[b, pl.ds(h * CHUNK, CHUNK)]

    pltpu.sync_copy(ids_hbm.at[pl.ds(l0, L_PER_W)], idx_v.at[0])

    gathers = [None] * N_CHUNK
    stores = [None] * N_CHUNK
    gathers[0] = pltpu.async_copy(tab_hbm.at[idx_slice(0)], bufs[0], gsems[0])
    gathers[1] = pltpu.async_copy(tab_hbm.at[idx_slice(1)], bufs[1], gsems[1])

    for b in range(1, B):
        pltpu.sync_copy(ids_hbm.at[pl.ds(b * L + l0, L_PER_W)], idx_v.at[b])
    pltpu.sync_copy(pos_hbm.at[pl.ds(l0, L_PER_W)], pos_v)
    pltpu.sync_copy(w_hbm, w_v)
    pltpu.sync_copy(bias_hbm, bias_v)

    def compute(gbuf, j):
        h = j % 2

        def sum_body(rr, carry):
            r0 = rr * 2
            r1 = r0 + 1
            p0 = h * CHUNK + r0
            p1 = p0 + 1

            z = jnp.zeros((LANES,), jnp.float32)

            @plsc.parallel_loop(0, NV, 2, unroll=4,
                                carry=(z, z, z, z, z, z, z, z))
            def slice_body(k, accs):
                a0, a1, a2, a3, q0, q1, q2, q3 = accs
                sl_e = pl.ds(k * LANES, LANES)
                sl_o = pl.ds((k + 1) * LANES, LANES)
                va = gbuf[r0, sl_e] + pos_v[p0, sl_e]
                vb = gbuf[r1, sl_e] + pos_v[p1, sl_e]
                vc = gbuf[r0, sl_o] + pos_v[p0, sl_o]
                vd = gbuf[r1, sl_o] + pos_v[p1, sl_o]
                gbuf[r0, sl_e] = va
                gbuf[r1, sl_e] = vb
                gbuf[r0, sl_o] = vc
                gbuf[r1, sl_o] = vd
                return (a0 + va, a1 + vc, a2 + vb, a3 + vd,
                        q0 + va * va, q1 + vc * vc,
                        q2 + vb * vb, q3 + vd * vd)

            a0, a1, a2, a3, q0, q1, q2, q3 = slice_body
            s0 = jnp.sum(a0 + a1)
            s1 = jnp.sum(a2 + a3)
            t0 = jnp.sum(q0 + q1)
            t1 = jnp.sum(q2 + q3)
            mean0 = s0 * (1.0 / H)
            mean1 = s1 * (1.0 / H)
            var0 = jnp.maximum(t0 * (1.0 / H) - mean0 * mean0, 0.0)
            var1 = jnp.maximum(t1 * (1.0 / H) - mean1 * mean1, 0.0)
            m_s[r0] = mean0
            m_s[r1] = mean1
            i_s[r0] = _rsqrt(var0 + EPS)
            i_s[r1] = _rsqrt(var1 + EPS)
            return carry

        lax.fori_loop(0, CHUNK // 2, sum_body, 0)

        def norm_slice(k, carry):
            sl = pl.ds(k * LANES, LANES)
            wv = w_v[sl]
            bv = bias_v[sl]

            def norm_rows(rg, c2):
                r0 = rg * 8
                for dr in range(8):
                    r = r0 + dr
                    gbuf[r, sl] = (gbuf[r, sl] - m_s[r]) * i_s[r] * wv + bv
                return c2

            lax.fori_loop(0, CHUNK // 8, norm_rows, 0)
            return carry

        lax.fori_loop(0, NV, norm_slice, 0)

    for j in range(N_CHUNK):
        bi = j % NBUF
        gathers[j].wait()
        compute(bufs[bi], j)
        b, h = j // 2, j % 2
        row0 = b * L + l0 + h * CHUNK
        stores[j] = pltpu.async_copy(bufs[bi], out_hbm.at[pl.ds(row0, CHUNK)],
                                     ssems[bi])
        nj = j + 2
        if nj < N_CHUNK:
            nbi = nj % NBUF
            if nj - NBUF >= 0:
                stores[nj - NBUF].wait()
            gathers[nj] = pltpu.async_copy(
                tab_hbm.at[idx_slice(nj)], bufs[nbi], gsems[nbi])
    for j in range(N_CHUNK - NBUF, N_CHUNK):
        stores[j].wait()


def kernel(input_ids, attention_mask, word_embeddings, position_embeddings,
           ln_weight, ln_bias):
    del attention_mask
    ids_flat = input_ids.reshape(-1).astype(jnp.int32)
    mesh = plsc.VectorSubcoreMesh(
        core_axis_name="c", subcore_axis_name="s",
        num_cores=NC, num_subcores=NS)
    fn = functools.partial(
        pl.kernel,
        out_type=jax.ShapeDtypeStruct((B * L, H), jnp.float32),
        mesh=mesh,
        scratch_types=[
            pltpu.VMEM((B, L_PER_W), jnp.int32),
            pltpu.VMEM((L_PER_W, H), jnp.float32),
            pltpu.VMEM((CHUNK, H), jnp.float32),
            pltpu.VMEM((CHUNK, H), jnp.float32),
            pltpu.VMEM((CHUNK, H), jnp.float32),
            pltpu.VMEM((H,), jnp.float32),
            pltpu.VMEM((H,), jnp.float32),
            pltpu.SMEM((CHUNK,), jnp.float32),
            pltpu.SMEM((CHUNK,), jnp.float32),
            pltpu.SemaphoreType.DMA,
            pltpu.SemaphoreType.DMA,
            pltpu.SemaphoreType.DMA,
            pltpu.SemaphoreType.DMA,
            pltpu.SemaphoreType.DMA,
            pltpu.SemaphoreType.DMA,
        ],
        compiler_params=pltpu.CompilerParams(needs_layout_passes=False),
    )(_body)
    out = fn(ids_flat, position_embeddings, word_embeddings, ln_weight, ln_bias)
    return out.reshape(B, L, H)
